# baseline (speedup 1.0000x reference)
import functools

import jax
import jax.numpy as jnp
from jax.experimental import pallas as pl
from jax.experimental.pallas import tpu as pltpu

_K = 100
_C = 16


def _body(cand_ref, ub_ref, w1_ref, b1_ref, w2t_ref, out_ref):
    Bb, S, D = ub_ref.shape
    c_all = cand_ref[...]

    W1 = w1_ref[...]
    Wc = W1[0:D] + W1[2 * D:3 * D]
    Wx = W1[D:2 * D] - W1[2 * D:3 * D]
    Wp = W1[3 * D:4 * D]
    b1 = b1_ref[...]
    w2t = w2t_ref[...]

    rows = jax.lax.broadcasted_iota(jnp.int32, (_C, _C * S), 0)
    cols = jax.lax.broadcasted_iota(jnp.int32, (_C, _C * S), 1)
    bmask = jnp.where(cols // S == rows, 1.0, 0.0)

    qs = []
    ws = []
    for j in range(Bb // _C):
        sl = slice(j * _C, (j + 1) * _C)
        X = ub_ref[sl]
        c = c_all[sl]
        prod = X * c[:, None, :]
        dot = jnp.sum(prod, axis=-1)
        n2sq = jnp.sum(X * X, axis=-1)
        qs.append(dot * jax.lax.rsqrt(jnp.maximum(n2sq, 1e-30)))
        ct2 = jnp.dot(c, Wc, preferred_element_type=jnp.float32) + b1
        M = (jnp.dot(prod.reshape(_C * S, D), Wp,
                     preferred_element_type=jnp.float32) +
             jnp.dot(X.reshape(_C * S, D), Wx,
                     preferred_element_type=jnp.float32))
        h = jnp.maximum(M.reshape(_C, S, -1) + ct2[:, None, :], 0.0)
        ws.append(jnp.sum(h * w2t[None], axis=-1))
    q = jnp.concatenate(qs, axis=0)
    w_all = jnp.concatenate(ws, axis=0)

    sb = jax.lax.bitcast_convert_type(q, jnp.uint32)
    neg = sb >> jnp.uint32(31)
    key = jnp.where(neg == jnp.uint32(1),
                    jnp.uint32(0xFFFFFFFF) - sb,
                    sb | jnp.uint32(0x80000000))
    r = jnp.zeros((Bb, 1), dtype=jnp.uint32)
    for bit in range(31, -1, -1):
        t = r | jnp.uint32(1 << bit)
        cnt = jnp.sum(jnp.where(key >= t, 1.0, 0.0), axis=-1, keepdims=True)
        r = jnp.where(cnt >= float(_K), t, r)
    mask = key >= r

    wl = jnp.where(mask, w_all, -1e30)
    m = jnp.max(wl, axis=-1, keepdims=True)
    e = jnp.exp(wl - m)
    p_all = e * (1.0 / jnp.sum(e, axis=-1, keepdims=True))
    for j in range(Bb // _C):
        sl = slice(j * _C, (j + 1) * _C)
        p = p_all[sl]
        P = jnp.tile(p, (1, _C)) * bmask
        out_ref[sl] = jnp.dot(P, ub_ref[sl].reshape(_C * S, D),
                              preferred_element_type=jnp.float32)


@jax.jit
def kernel(candidate, user_behavior, W1, b1, W2, b2):
    B, S, D = user_behavior.shape
    Bb = 64
    f = pl.pallas_call(
        _body,
        grid=(B // Bb,),
        in_specs=[
            pl.BlockSpec((Bb, D), lambda i: (i, 0)),
            pl.BlockSpec((Bb, S, D), lambda i: (i, 0, 0)),
            pl.BlockSpec((4 * D, W1.shape[1]), lambda i: (0, 0)),
            pl.BlockSpec((1, b1.shape[0]), lambda i: (0, 0)),
            pl.BlockSpec((1, W2.shape[0]), lambda i: (0, 0)),
        ],
        out_specs=pl.BlockSpec((Bb, D), lambda i: (i, 0)),
        out_shape=jax.ShapeDtypeStruct((B, D), jnp.float32),
        compiler_params=pltpu.CompilerParams(
            dimension_semantics=("parallel",)),
    )
    return f(candidate, user_behavior, W1, b1.reshape(1, -1), W2.T)


# speedup vs baseline: 1.1231x; 1.1231x over previous
import functools

import jax
import jax.numpy as jnp
from jax.experimental import pallas as pl
from jax.experimental.pallas import tpu as pltpu

_K = 100
_C = 16


def _body(cand_ref, ub_ref, w1_ref, b1_ref, w2t_ref, out_ref):
    Bb, S, D = ub_ref.shape
    c_all = cand_ref[...]

    W1 = w1_ref[...]
    Wc = W1[0:D] + W1[2 * D:3 * D]
    Wx = W1[D:2 * D] - W1[2 * D:3 * D]
    Wp = W1[3 * D:4 * D]
    b1 = b1_ref[...]
    w2t = w2t_ref[...]

    rows = jax.lax.broadcasted_iota(jnp.int32, (_C, _C * S), 0)
    cols = jax.lax.broadcasted_iota(jnp.int32, (_C, _C * S), 1)
    bmask = jnp.where(cols // S == rows, 1.0, 0.0)

    qs = []
    ws = []
    for j in range(Bb // _C):
        sl = slice(j * _C, (j + 1) * _C)
        X = ub_ref[sl]
        c = c_all[sl]
        prod = X * c[:, None, :]
        dot = jnp.sum(prod, axis=-1)
        n2sq = jnp.sum(X * X, axis=-1)
        qs.append(dot * jax.lax.rsqrt(jnp.maximum(n2sq, 1e-30)))
        ct2 = jnp.dot(c, Wc, preferred_element_type=jnp.float32) + b1
        M = (jnp.dot(prod.reshape(_C * S, D), Wp,
                     preferred_element_type=jnp.float32) +
             jnp.dot(X.reshape(_C * S, D), Wx,
                     preferred_element_type=jnp.float32))
        h = jnp.maximum(M.reshape(_C, S, -1) + ct2[:, None, :], 0.0)
        ws.append(jnp.sum(h * w2t[None], axis=-1))
    q = jnp.concatenate(qs, axis=0)
    w_all = jnp.concatenate(ws, axis=0)

    sb = jax.lax.bitcast_convert_type(q, jnp.uint32)
    neg = sb >> jnp.uint32(31)
    key = jnp.where(neg == jnp.uint32(1),
                    jnp.uint32(0xFFFFFFFF) - sb,
                    sb | jnp.uint32(0x80000000))
    r = jnp.zeros((Bb, 1), dtype=jnp.uint32)
    for bit in range(31, -1, -1):
        t = r | jnp.uint32(1 << bit)
        cnt = jnp.sum(jnp.where(key >= t, 1.0, 0.0), axis=-1, keepdims=True)
        r = jnp.where(cnt >= float(_K), t, r)
    mask = key >= r

    wl = jnp.where(mask, w_all, -1e30)
    m = jnp.max(wl, axis=-1, keepdims=True)
    e = jnp.exp(wl - m)
    p_all = e * (1.0 / jnp.sum(e, axis=-1, keepdims=True))
    for j in range(Bb // _C):
        sl = slice(j * _C, (j + 1) * _C)
        p = p_all[sl]
        P = jnp.tile(p, (1, _C)) * bmask
        out_ref[sl] = jnp.dot(P, ub_ref[sl].reshape(_C * S, D),
                              preferred_element_type=jnp.float32)


@jax.jit
def kernel(candidate, user_behavior, W1, b1, W2, b2):
    B, S, D = user_behavior.shape
    Bb = 256
    f = pl.pallas_call(
        _body,
        grid=(B // Bb,),
        in_specs=[
            pl.BlockSpec((Bb, D), lambda i: (i, 0)),
            pl.BlockSpec((Bb, S, D), lambda i: (i, 0, 0)),
            pl.BlockSpec((4 * D, W1.shape[1]), lambda i: (0, 0)),
            pl.BlockSpec((1, b1.shape[0]), lambda i: (0, 0)),
            pl.BlockSpec((1, W2.shape[0]), lambda i: (0, 0)),
        ],
        out_specs=pl.BlockSpec((Bb, D), lambda i: (i, 0)),
        out_shape=jax.ShapeDtypeStruct((B, D), jnp.float32),
        compiler_params=pltpu.CompilerParams(
            dimension_semantics=("parallel",)),
    )
    return f(candidate, user_behavior, W1, b1.reshape(1, -1), W2.T)


# speedup vs baseline: 1.1259x; 1.0025x over previous
import jax
import jax.numpy as jnp
from jax.experimental import pallas as pl
from jax.experimental.pallas import tpu as pltpu

_K = 100
_C = 16


def _body(cand_ref, ub_ref, w1_ref, b1_ref, w2t_ref, out_ref):
    Bb, S, D = ub_ref.shape
    c_all = cand_ref[...]

    W1 = w1_ref[...]
    Wc = W1[0:D] + W1[2 * D:3 * D]
    Wx = W1[D:2 * D] - W1[2 * D:3 * D]
    Wp = W1[3 * D:4 * D]
    b1 = b1_ref[...]
    w2t = w2t_ref[...]

    rows = jax.lax.broadcasted_iota(jnp.int32, (_C, _C * S), 0)
    cols = jax.lax.broadcasted_iota(jnp.int32, (_C, _C * S), 1)
    bmask = jnp.where(cols // S == rows, 1.0, 0.0)

    qs = []
    ws = []
    for j in range(Bb // _C):
        sl = slice(j * _C, (j + 1) * _C)
        X = ub_ref[sl]
        c = c_all[sl]
        prod = X * c[:, None, :]
        dot = jnp.sum(prod, axis=-1)
        n2sq = jnp.sum(X * X, axis=-1)
        qs.append(dot * jax.lax.rsqrt(jnp.maximum(n2sq, 1e-30)))
        ct2 = jnp.dot(c, Wc, preferred_element_type=jnp.float32) + b1
        M = (jnp.dot(prod.reshape(_C * S, D), Wp,
                     preferred_element_type=jnp.float32) +
             jnp.dot(X.reshape(_C * S, D), Wx,
                     preferred_element_type=jnp.float32))
        h = jnp.maximum(M.reshape(_C, S, -1) + ct2[:, None, :], 0.0)
        ws.append(jnp.sum(h * w2t[None], axis=-1))
    q = jnp.concatenate(qs, axis=0)
    w_all = jnp.concatenate(ws, axis=0)

    sb = jax.lax.bitcast_convert_type(q, jnp.uint32)
    neg = sb >> jnp.uint32(31)
    key = jnp.where(neg == jnp.uint32(1),
                    jnp.uint32(0xFFFFFFFF) - sb,
                    sb | jnp.uint32(0x80000000))
    r = jnp.zeros((Bb, 1), dtype=jnp.uint32)
    for bit in range(31, -1, -1):
        t = r | jnp.uint32(1 << bit)
        cnt = jnp.sum(jnp.where(key >= t, 1.0, 0.0), axis=-1, keepdims=True)
        r = jnp.where(cnt >= float(_K), t, r)
    mask = key >= r

    wl = jnp.where(mask, w_all, -1e30)
    m = jnp.max(wl, axis=-1, keepdims=True)
    e = jnp.exp(wl - m)
    p_all = e * (1.0 / jnp.sum(e, axis=-1, keepdims=True))
    for j in range(Bb // _C):
        sl = slice(j * _C, (j + 1) * _C)
        p = p_all[sl]
        P = jnp.tile(p, (1, _C)) * bmask
        out_ref[sl] = jnp.dot(P, ub_ref[sl].reshape(_C * S, D),
                              preferred_element_type=jnp.float32)


@jax.jit
def kernel(candidate, user_behavior, W1, b1, W2, b2):
    B, S, D = user_behavior.shape
    Bb = 256
    f = pl.pallas_call(
        _body,
        grid=(B // Bb,),
        in_specs=[
            pl.BlockSpec((Bb, D), lambda i: (i, 0)),
            pl.BlockSpec((Bb, S, D), lambda i: (i, 0, 0)),
            pl.BlockSpec((4 * D, W1.shape[1]), lambda i: (0, 0)),
            pl.BlockSpec((1, b1.shape[0]), lambda i: (0, 0)),
            pl.BlockSpec((1, W2.shape[0]), lambda i: (0, 0)),
        ],
        out_specs=pl.BlockSpec((Bb, D), lambda i: (i, 0)),
        out_shape=jax.ShapeDtypeStruct((B, D), jnp.float32),
        compiler_params=pltpu.CompilerParams(
            dimension_semantics=("parallel",)),
    )
    return f(candidate, user_behavior, W1, b1.reshape(1, -1), W2.T)
